# baseline (speedup 1.0000x reference)
import jax
import jax.numpy as jnp
from jax.experimental import pallas as pl
from jax.experimental.pallas import tpu as pltpu

T = 7
N = 512
RNN = 256
EMB = 128
OUT = 5
NTS = 32
K = 128
V = 64

BN = 512
NB = N // BN


def _fused(nodes_ref, ttc_ref, ttc2_ref, veh_ref, m_ref,
           win_ref, bin_ref, wt_ref, bt_ref, wtv_ref, btv_ref,
           wih_ref, whh_ref, bg_ref, wout_ref, bout_ref,
           h0_ref, c0_ref,
           out_ref, hs_ref, cs_ref,
           sv_ref):
    f = pl.program_id(0)
    nb = pl.program_id(1)
    n0 = nb * BN

    @pl.when((f == 0) & (nb == 0))
    def _():
        hs_ref[...] = h0_ref[...]
        cs_ref[...] = c0_ref[...]

    @pl.when(nb == 0)
    def _():
        sv_ref[...] = jnp.max(veh_ref[0], axis=0)

    social = jnp.concatenate(
        [jnp.max(ttc_ref[0], axis=2),
         jnp.max(ttc2_ref[0], axis=2)], axis=0)
    svb = sv_ref[:, pl.ds(n0, BN)]

    nodes = nodes_ref[f, pl.ds(n0, BN), :]
    inp_emb = jax.nn.relu(
        jnp.dot(nodes, win_ref[...], preferred_element_type=jnp.float32)
        + bin_ref[...])
    t_emb = jax.nn.relu(
        jnp.dot(social, wt_ref[...], preferred_element_type=jnp.float32)
        + bt_ref[...])
    tv_emb = jax.nn.relu(
        jax.lax.dot_general(svb, wtv_ref[...], (((0,), (0,)), ((), ())),
                            preferred_element_type=jnp.float32)
        + btv_ref[...])
    concat = jnp.concatenate([inp_emb, t_emb, tv_emb], axis=1)

    h = hs_ref[pl.ds(n0, BN), :]
    c = cs_ref[pl.ds(n0, BN), :]
    gates = (jnp.dot(concat, wih_ref[...], preferred_element_type=jnp.float32)
             + jnp.dot(h, whh_ref[...], preferred_element_type=jnp.float32)
             + bg_ref[...])
    i_g = jax.nn.sigmoid(gates[:, 0:RNN])
    f_g = jax.nn.sigmoid(gates[:, RNN:2 * RNN])
    g_g = jnp.tanh(gates[:, 2 * RNN:3 * RNN])
    o_g = jax.nn.sigmoid(gates[:, 3 * RNN:4 * RNN])
    c_new = f_g * c + i_g * g_g
    h_new = o_g * jnp.tanh(c_new)

    out_raw = (jnp.dot(h_new, wout_ref[...], preferred_element_type=jnp.float32)
               + bout_ref[...])

    m = m_ref[f, pl.ds(n0, BN), :]
    out_ref[f, pl.ds(n0, BN), :] = m * out_raw
    hs_ref[pl.ds(n0, BN), :] = h + m * (h_new - h)
    cs_ref[pl.ds(n0, BN), :] = c + m * (c_new - c)


def kernel(input_data, grids, hidden_states, cell_states, mask, input_data_veh,
           grids_veh, mask_veh, grids_TTC, grids_TTC_veh,
           W_in, b_in, W_t, b_t, W_tv, b_tv, W_ih, W_hh, b_ih, b_hh,
           W_out, b_out):
    del grids, input_data_veh, grids_veh, mask_veh

    ttc_t = jnp.transpose(grids_TTC, (0, 1, 3, 2))
    veh_t = jnp.transpose(grids_TTC_veh, (0, 2, 3, 1))
    maskf = mask.astype(jnp.float32).reshape(T, N, 1)

    win = W_in.T
    wt = W_t.T
    wtv = W_tv.T
    wih = W_ih.T
    whh = W_hh.T
    bg = (b_ih + b_hh).reshape(1, 4 * RNN)
    wout = W_out.T
    bout = b_out.reshape(1, OUT)
    bin2 = b_in.reshape(1, EMB)
    bt2 = b_t.reshape(1, EMB)
    btv2 = b_tv.reshape(1, EMB)

    grid = (T, NB)

    def whole(shape):
        nd = len(shape)
        return pl.BlockSpec(shape, lambda f, nb, _nd=nd: (0,) * _nd)

    outputs, hs, cs = pl.pallas_call(
        _fused,
        grid=grid,
        in_specs=[
            whole((T, N, 2)),
            pl.BlockSpec((1, BN // 2, NTS, K), lambda f, nb: (f, 0, 0, 0)),
            pl.BlockSpec((1, BN // 2, NTS, K), lambda f, nb: (f, 1, 0, 0)),
            pl.BlockSpec((1, V, NTS, N), lambda f, nb: (f, 0, 0, 0)),
            whole((T, N, 1)),
            whole((2, EMB)),
            whole((1, EMB)),
            whole((NTS, EMB)),
            whole((1, EMB)),
            whole((NTS, EMB)),
            whole((1, EMB)),
            whole((3 * EMB, 4 * RNN)),
            whole((RNN, 4 * RNN)),
            whole((1, 4 * RNN)),
            whole((RNN, OUT)),
            whole((1, OUT)),
            whole((N, RNN)),
            whole((N, RNN)),
        ],
        out_specs=[
            whole((T, N, OUT)),
            whole((N, RNN)),
            whole((N, RNN)),
        ],
        out_shape=[
            jax.ShapeDtypeStruct((T, N, OUT), jnp.float32),
            jax.ShapeDtypeStruct((N, RNN), jnp.float32),
            jax.ShapeDtypeStruct((N, RNN), jnp.float32),
        ],
        scratch_shapes=[
            pltpu.VMEM((NTS, N), jnp.float32),
        ],
        compiler_params=pltpu.CompilerParams(
            dimension_semantics=("arbitrary", "arbitrary"),
        ),
    )(input_data, ttc_t, ttc_t, veh_t, maskf,
      win, bin2, wt, bt2, wtv, btv2, wih, whh, bg, wout, bout,
      hidden_states, cell_states)

    return outputs, hs, cs


# speedup vs baseline: 1.0267x; 1.0267x over previous
import jax
import jax.numpy as jnp
from jax.experimental import pallas as pl
from jax.experimental.pallas import tpu as pltpu

T = 7
N = 512
RNN = 256
EMB = 128
OUT = 5
NTS = 32
K = 128
V = 64

TSPLIT = 2
NH = N // TSPLIT


def _fused(nodes_ref, ttc_hbm, veh_hbm, m_ref,
           win_ref, bin_ref, wt_ref, bt_ref, wtv_ref, btv_ref,
           wih_ref, whh_ref, bg_ref, wout_ref, bout_ref,
           h0_ref, c0_ref,
           out_ref, hs_ref, cs_ref,
           ttc_buf, veh_buf, ttc_sem, veh_sem):
    f = pl.program_id(0)

    def copies(f2, b2):
        cps = []
        for q in range(TSPLIT):
            cps.append(pltpu.make_async_copy(
                ttc_hbm.at[pl.ds(f2, 1), pl.ds(q * NH, NH), :, :],
                ttc_buf.at[pl.ds(b2, 1), pl.ds(q * NH, NH), :, :],
                ttc_sem.at[b2, q]))
        cps.append(pltpu.make_async_copy(
            veh_hbm.at[pl.ds(f2, 1), :, :, :],
            veh_buf.at[pl.ds(b2, 1), :, :, :],
            veh_sem.at[b2]))
        return cps

    @pl.when(f == 0)
    def _():
        hs_ref[...] = h0_ref[...]
        cs_ref[...] = c0_ref[...]
        for cp in copies(0, 0):
            cp.start()

    @pl.when(f + 1 < T)
    def _():
        for cp in copies(f + 1, (f + 1) % 2):
            cp.start()

    buf = f % 2
    for cp in copies(f, buf):
        cp.wait()

    social = jnp.max(ttc_buf[buf], axis=2)
    sv = jnp.max(veh_buf[buf], axis=0)

    nodes = nodes_ref[f]
    inp_emb = jax.nn.relu(
        jnp.dot(nodes, win_ref[...], preferred_element_type=jnp.float32)
        + bin_ref[...])
    t_emb = jax.nn.relu(
        jnp.dot(social, wt_ref[...], preferred_element_type=jnp.float32)
        + bt_ref[...])
    tv_emb = jax.nn.relu(
        jax.lax.dot_general(sv, wtv_ref[...], (((0,), (0,)), ((), ())),
                            preferred_element_type=jnp.float32)
        + btv_ref[...])
    concat = jnp.concatenate([inp_emb, t_emb, tv_emb], axis=1)

    h = hs_ref[...]
    c = cs_ref[...]
    gates = (jnp.dot(concat, wih_ref[...], preferred_element_type=jnp.float32)
             + jnp.dot(h, whh_ref[...], preferred_element_type=jnp.float32)
             + bg_ref[...])
    i_g = jax.nn.sigmoid(gates[:, 0:RNN])
    f_g = jax.nn.sigmoid(gates[:, RNN:2 * RNN])
    g_g = jnp.tanh(gates[:, 2 * RNN:3 * RNN])
    o_g = jax.nn.sigmoid(gates[:, 3 * RNN:4 * RNN])
    c_new = f_g * c + i_g * g_g
    h_new = o_g * jnp.tanh(c_new)

    out_raw = (jnp.dot(h_new, wout_ref[...], preferred_element_type=jnp.float32)
               + bout_ref[...])

    m = m_ref[f]
    out_ref[f] = m * out_raw
    hs_ref[...] = h + m * (h_new - h)
    cs_ref[...] = c + m * (c_new - c)


def kernel(input_data, grids, hidden_states, cell_states, mask, input_data_veh,
           grids_veh, mask_veh, grids_TTC, grids_TTC_veh,
           W_in, b_in, W_t, b_t, W_tv, b_tv, W_ih, W_hh, b_ih, b_hh,
           W_out, b_out):
    del grids, input_data_veh, grids_veh, mask_veh

    ttc_t = jnp.transpose(grids_TTC, (0, 1, 3, 2))
    veh_t = jnp.transpose(grids_TTC_veh, (0, 2, 3, 1))
    maskf = mask.astype(jnp.float32).reshape(T, N, 1)

    win = W_in.T
    wt = W_t.T
    wtv = W_tv.T
    wih = W_ih.T
    whh = W_hh.T
    bg = (b_ih + b_hh).reshape(1, 4 * RNN)
    wout = W_out.T
    bout = b_out.reshape(1, OUT)
    bin2 = b_in.reshape(1, EMB)
    bt2 = b_t.reshape(1, EMB)
    btv2 = b_tv.reshape(1, EMB)

    def whole(shape):
        nd = len(shape)
        return pl.BlockSpec(shape, lambda f, _nd=nd: (0,) * _nd)

    any_spec = pl.BlockSpec(memory_space=pltpu.MemorySpace.HBM)

    outputs, hs, cs = pl.pallas_call(
        _fused,
        grid=(T,),
        in_specs=[
            whole((T, N, 2)),
            any_spec,
            any_spec,
            whole((T, N, 1)),
            whole((2, EMB)),
            whole((1, EMB)),
            whole((NTS, EMB)),
            whole((1, EMB)),
            whole((NTS, EMB)),
            whole((1, EMB)),
            whole((3 * EMB, 4 * RNN)),
            whole((RNN, 4 * RNN)),
            whole((1, 4 * RNN)),
            whole((RNN, OUT)),
            whole((1, OUT)),
            whole((N, RNN)),
            whole((N, RNN)),
        ],
        out_specs=[
            whole((T, N, OUT)),
            whole((N, RNN)),
            whole((N, RNN)),
        ],
        out_shape=[
            jax.ShapeDtypeStruct((T, N, OUT), jnp.float32),
            jax.ShapeDtypeStruct((N, RNN), jnp.float32),
            jax.ShapeDtypeStruct((N, RNN), jnp.float32),
        ],
        scratch_shapes=[
            pltpu.VMEM((2, N, NTS, K), jnp.float32),
            pltpu.VMEM((2, V, NTS, N), jnp.float32),
            pltpu.SemaphoreType.DMA((2, TSPLIT)),
            pltpu.SemaphoreType.DMA((2,)),
        ],
        compiler_params=pltpu.CompilerParams(
            dimension_semantics=("arbitrary",),
        ),
    )(input_data, ttc_t, veh_t, maskf,
      win, bin2, wt, bt2, wtv, btv2, wih, whh, bg, wout, bout,
      hidden_states, cell_states)

    return outputs, hs, cs


# speedup vs baseline: 1.0302x; 1.0034x over previous
import jax
import jax.numpy as jnp
from jax.experimental import pallas as pl
from jax.experimental.pallas import tpu as pltpu

T = 7
N = 512
RNN = 256
EMB = 128
OUT = 5
NTS = 32
K = 128
V = 64

TSPLIT = 4
NH = N // TSPLIT


def _fused(nodes_ref, ttc_hbm, veh_hbm, m_ref,
           win_ref, bin_ref, wt_ref, bt_ref, wtv_ref, btv_ref,
           wih_ref, whh_ref, bg_ref, wout_ref, bout_ref,
           h0_ref, c0_ref,
           out_ref, hs_ref, cs_ref,
           ttc_buf, veh_buf, ttc_sem, veh_sem):
    f = pl.program_id(0)

    def copies(f2, b2):
        cps = []
        for q in range(TSPLIT):
            cps.append(pltpu.make_async_copy(
                ttc_hbm.at[pl.ds(f2, 1), pl.ds(q * NH, NH), :, :],
                ttc_buf.at[pl.ds(b2, 1), pl.ds(q * NH, NH), :, :],
                ttc_sem.at[b2, q]))
        cps.append(pltpu.make_async_copy(
            veh_hbm.at[pl.ds(f2, 1), :, :, :],
            veh_buf.at[pl.ds(b2, 1), :, :, :],
            veh_sem.at[b2]))
        return cps

    @pl.when(f == 0)
    def _():
        hs_ref[...] = h0_ref[...]
        cs_ref[...] = c0_ref[...]
        for j in range(2):
            for cp in copies(j, j):
                cp.start()

    @pl.when(f + 2 < T)
    def _():
        for cp in copies(f + 2, (f + 2) % 3):
            cp.start()

    buf = f % 3
    for cp in copies(f, buf):
        cp.wait()

    social = jnp.max(ttc_buf[buf], axis=2)
    sv = jnp.max(veh_buf[buf], axis=0)

    nodes = nodes_ref[f]
    inp_emb = jax.nn.relu(
        jnp.dot(nodes, win_ref[...], preferred_element_type=jnp.float32)
        + bin_ref[...])
    t_emb = jax.nn.relu(
        jnp.dot(social, wt_ref[...], preferred_element_type=jnp.float32)
        + bt_ref[...])
    tv_emb = jax.nn.relu(
        jax.lax.dot_general(sv, wtv_ref[...], (((0,), (0,)), ((), ())),
                            preferred_element_type=jnp.float32)
        + btv_ref[...])
    concat = jnp.concatenate([inp_emb, t_emb, tv_emb], axis=1)

    h = hs_ref[...]
    c = cs_ref[...]
    gates = (jnp.dot(concat, wih_ref[...], preferred_element_type=jnp.float32)
             + jnp.dot(h, whh_ref[...], preferred_element_type=jnp.float32)
             + bg_ref[...])
    i_g = jax.nn.sigmoid(gates[:, 0:RNN])
    f_g = jax.nn.sigmoid(gates[:, RNN:2 * RNN])
    g_g = jnp.tanh(gates[:, 2 * RNN:3 * RNN])
    o_g = jax.nn.sigmoid(gates[:, 3 * RNN:4 * RNN])
    c_new = f_g * c + i_g * g_g
    h_new = o_g * jnp.tanh(c_new)

    out_raw = (jnp.dot(h_new, wout_ref[...], preferred_element_type=jnp.float32)
               + bout_ref[...])

    m = m_ref[f]
    out_ref[f] = m * out_raw
    hs_ref[...] = h + m * (h_new - h)
    cs_ref[...] = c + m * (c_new - c)


def kernel(input_data, grids, hidden_states, cell_states, mask, input_data_veh,
           grids_veh, mask_veh, grids_TTC, grids_TTC_veh,
           W_in, b_in, W_t, b_t, W_tv, b_tv, W_ih, W_hh, b_ih, b_hh,
           W_out, b_out):
    del grids, input_data_veh, grids_veh, mask_veh

    ttc_t = jnp.transpose(grids_TTC, (0, 1, 3, 2))
    veh_t = jnp.transpose(grids_TTC_veh, (0, 2, 3, 1))
    maskf = mask.astype(jnp.float32).reshape(T, N, 1)

    win = W_in.T
    wt = W_t.T
    wtv = W_tv.T
    wih = W_ih.T
    whh = W_hh.T
    bg = (b_ih + b_hh).reshape(1, 4 * RNN)
    wout = W_out.T
    bout = b_out.reshape(1, OUT)
    bin2 = b_in.reshape(1, EMB)
    bt2 = b_t.reshape(1, EMB)
    btv2 = b_tv.reshape(1, EMB)

    def whole(shape):
        nd = len(shape)
        return pl.BlockSpec(shape, lambda f, _nd=nd: (0,) * _nd)

    any_spec = pl.BlockSpec(memory_space=pltpu.MemorySpace.HBM)

    outputs, hs, cs = pl.pallas_call(
        _fused,
        grid=(T,),
        in_specs=[
            whole((T, N, 2)),
            any_spec,
            any_spec,
            whole((T, N, 1)),
            whole((2, EMB)),
            whole((1, EMB)),
            whole((NTS, EMB)),
            whole((1, EMB)),
            whole((NTS, EMB)),
            whole((1, EMB)),
            whole((3 * EMB, 4 * RNN)),
            whole((RNN, 4 * RNN)),
            whole((1, 4 * RNN)),
            whole((RNN, OUT)),
            whole((1, OUT)),
            whole((N, RNN)),
            whole((N, RNN)),
        ],
        out_specs=[
            whole((T, N, OUT)),
            whole((N, RNN)),
            whole((N, RNN)),
        ],
        out_shape=[
            jax.ShapeDtypeStruct((T, N, OUT), jnp.float32),
            jax.ShapeDtypeStruct((N, RNN), jnp.float32),
            jax.ShapeDtypeStruct((N, RNN), jnp.float32),
        ],
        scratch_shapes=[
            pltpu.VMEM((3, N, NTS, K), jnp.float32),
            pltpu.VMEM((3, V, NTS, N), jnp.float32),
            pltpu.SemaphoreType.DMA((3, TSPLIT)),
            pltpu.SemaphoreType.DMA((3,)),
        ],
        compiler_params=pltpu.CompilerParams(
            dimension_semantics=("arbitrary",),
        ),
    )(input_data, ttc_t, veh_t, maskf,
      win, bin2, wt, bt2, wtv, btv2, wih, whh, bg, wout, bout,
      hidden_states, cell_states)

    return outputs, hs, cs
